# baseline (speedup 1.0000x reference)
import functools

import jax
import jax.numpy as jnp
from jax import lax
from jax.experimental import pallas as pl
from jax.experimental.pallas import tpu as pltpu
from jax.experimental.pallas import tpu_sc as plsc

NC = 2
NS = 16
L = 16
NW = NC * NS

B = 16384
K = 10
D = 64
VOCAB = 1000000

BPW = B // NW
CHUNK = 32
NCHUNK = BPW // CHUNK
NGRP = CHUNK // L
NSROWS = CHUNK * K
NSU = 4
NSUR = NSROWS // NSU

NBLK = VOCAB // 128
TPW = NBLK // NW
NEXTRA = NBLK - TPW * NW
TAILROW = VOCAB - 128


def _transpose_block(in_v, out_v, iota):
  rowv = [ci * L + iota for ci in range(128 // L)]
  for d in range(D):
    dv = jnp.full((L,), d, jnp.int32)
    for ci in range(128 // L):
      v = in_v[d, pl.ds(ci * L, L)]
      plsc.store_scatter(out_v, [rowv[ci], dv], v)


def _tr_body(wt_hbm, tail_hbm, out_hbm,
             in0, in1, out0, out1, semi0, semi1, semo0, semo1):
  wid = lax.axis_index("s") * NC + lax.axis_index("c")
  iota = lax.iota(jnp.int32, L)
  ins = (in0, in1)
  outs = (out0, out1)
  isems = (semi0, semi1)
  osems = (semo0, semo1)

  def rd(j, b):
    pltpu.async_copy(wt_hbm.at[:, pl.ds(j * 128, 128)], ins[b], isems[b])

  def wr(j, b):
    pltpu.async_copy(outs[b], out_hbm.at[pl.ds(j * 128, 128)], osems[b])

  def wait_rd(b):
    pltpu.make_async_copy(wt_hbm.at[:, pl.ds(0, 128)], ins[b],
                          isems[b]).wait()

  def wait_wr(b):
    pltpu.make_async_copy(outs[b], out_hbm.at[pl.ds(0, 128)],
                          osems[b]).wait()

  jof = lambda t: wid + NW * t
  jclamp = lambda t: jnp.minimum(jof(t), NBLK - 1)

  rd(jof(0), 0)
  rd(jof(1), 1)
  for b in range(2):
    wait_rd(b)
    _transpose_block(ins[b], outs[b], iota)
    rd(jclamp(b + 2), b)
    wr(jof(b), b)

  def body(s, carry):
    for b in range(2):
      t = 2 * s + b
      wait_rd(b)
      wait_wr(b)
      _transpose_block(ins[b], outs[b], iota)
      rd(jclamp(t + 2), b)
      wr(jof(t), b)
    return carry

  lax.fori_loop(1, TPW // 2, body, 0)

  for b in range(2):
    wait_rd(b)
    wait_wr(b)

  @pl.when(wid < NEXTRA)
  def _():
    j = TPW * NW + wid
    pltpu.async_copy(wt_hbm.at[:, pl.ds(j * 128, 128)], ins[0],
                     isems[0]).wait()
    _transpose_block(ins[0], outs[0], iota)
    pltpu.async_copy(outs[0], out_hbm.at[pl.ds(j * 128, 128)],
                     osems[0]).wait()

  @pl.when(wid == NEXTRA)
  def _():
    pltpu.async_copy(tail_hbm, ins[1], isems[1]).wait()
    _transpose_block(ins[1], outs[1], iota)
    pltpu.async_copy(outs[1], out_hbm.at[pl.ds(TAILROW, 128)],
                     osems[1]).wait()


def _retile(wt, tail):
  mesh = plsc.VectorSubcoreMesh(core_axis_name="c", subcore_axis_name="s")
  return pl.kernel(
      _tr_body,
      out_type=jax.ShapeDtypeStruct((VOCAB, 2 * D), jnp.float32),
      mesh=mesh,
      compiler_params=pltpu.CompilerParams(needs_layout_passes=False),
      scratch_types=[
          pltpu.VMEM((D, 128), jnp.float32),
          pltpu.VMEM((D, 128), jnp.float32),
          pltpu.VMEM((128, 128), jnp.float32),
          pltpu.VMEM((128, 128), jnp.float32),
          pltpu.SemaphoreType.DMA,
          pltpu.SemaphoreType.DMA,
          pltpu.SemaphoreType.DMA,
          pltpu.SemaphoreType.DMA,
      ],
  )(wt, tail)


def _sc_body(cen_i, ctx_i, ns_i, wc_hbm, wx_hbm, out_hbm,
             icen_v, ictx_v, ins_v, score_v,
             c_rows0, c_rows1, x_rows0, x_rows1, n_rows0, n_rows1,
             sem0, sem1):
  wid = lax.axis_index("s") * NC + lax.axis_index("c")

  pltpu.sync_copy(cen_i.at[wid], icen_v)
  pltpu.sync_copy(ctx_i.at[wid], ictx_v)
  pltpu.sync_copy(ns_i.at[wid], ins_v)

  bufs = ((c_rows0, x_rows0, n_rows0, sem0),
          (c_rows1, x_rows1, n_rows1, sem1))

  def issue(g):
    c_b, x_b, n_b, sem = bufs[g % 2]
    cps = [
        pltpu.async_copy(wc_hbm.at[icen_v.at[pl.ds(g * CHUNK, CHUNK)]],
                         c_b, sem),
        pltpu.async_copy(wx_hbm.at[ictx_v.at[pl.ds(g * CHUNK, CHUNK)]],
                         x_b, sem),
    ]
    for u in range(NSU):
      cps.append(pltpu.async_copy(
          wx_hbm.at[ins_v.at[pl.ds(g * NSROWS + u * NSUR, NSUR)]],
          n_b.at[pl.ds(u * NSUR, NSUR)], sem))
    return cps

  iota = lax.iota(jnp.int32, L)
  pending = issue(0)

  for g in range(NCHUNK):
    nxt = issue(g + 1) if g + 1 < NCHUNK else None
    for cp in pending:
      cp.wait()
    pending = nxt

    c_b, x_b, n_b, _ = bufs[g % 2]
    for grp in range(NGRP):
      base = g * CHUNK + grp * L
      row = grp * L + iota
      nrow = [row * K + k for k in range(K)]

      def body(d, accs):
        dv = jnp.full((L,), d, jnp.int32)
        cv = plsc.load_gather(c_b, [row, dv])
        xv = plsc.load_gather(x_b, [row, dv])
        new = [accs[0] + cv * xv]
        for k in range(K):
          nv = plsc.load_gather(n_b, [nrow[k], dv])
          new.append(accs[k + 1] + cv * nv)
        return tuple(new)

      accs = lax.fori_loop(
          0, D, body, tuple(jnp.zeros((L,), jnp.float32) for _ in range(K + 1)))

      score_v[0, pl.ds(base, L)] = -accs[0]
      for k in range(K):
        score_v[1 + k, pl.ds(base, L)] = accs[k + 1]

  pltpu.sync_copy(score_v, out_hbm.at[wid])


def _tc_body(s_ref, o_ref):
  x = s_ref[...]
  o_ref[0, 0] = jnp.sum(jnp.maximum(x, 0.0) +
                        jnp.log1p(jnp.exp(-jnp.abs(x))))


@jax.jit
def kernel(center, context, ns, W_center, W_context):
  cen_i = center.astype(jnp.int32).reshape(NW, BPW)
  ctx_i = context.astype(jnp.int32).reshape(NW, BPW)
  ns_i = ns.astype(jnp.int32).reshape(NW, BPW * K)

  wct = W_center.T
  wxt = W_context.T
  wc2 = _retile(wct, lax.slice(wct, (0, TAILROW), (D, VOCAB)))
  wx2 = _retile(wxt, lax.slice(wxt, (0, TAILROW), (D, VOCAB)))

  mesh = plsc.VectorSubcoreMesh(core_axis_name="c", subcore_axis_name="s")
  scores = pl.kernel(
      _sc_body,
      out_type=jax.ShapeDtypeStruct((NW, 1 + K, BPW), jnp.float32),
      mesh=mesh,
      compiler_params=pltpu.CompilerParams(needs_layout_passes=False),
      scratch_types=[
          pltpu.VMEM((BPW,), jnp.int32),
          pltpu.VMEM((BPW,), jnp.int32),
          pltpu.VMEM((BPW * K,), jnp.int32),
          pltpu.VMEM((1 + K, BPW), jnp.float32),
          pltpu.VMEM((CHUNK, 2 * D), jnp.float32),
          pltpu.VMEM((CHUNK, 2 * D), jnp.float32),
          pltpu.VMEM((CHUNK, 2 * D), jnp.float32),
          pltpu.VMEM((CHUNK, 2 * D), jnp.float32),
          pltpu.VMEM((NSROWS, 2 * D), jnp.float32),
          pltpu.VMEM((NSROWS, 2 * D), jnp.float32),
          pltpu.SemaphoreType.DMA,
          pltpu.SemaphoreType.DMA,
      ],
  )(cen_i, ctx_i, ns_i, wc2, wx2)

  loss = pl.pallas_call(
      _tc_body,
      out_shape=jax.ShapeDtypeStruct((1, 1), jnp.float32),
      out_specs=pl.BlockSpec(memory_space=pltpu.SMEM),
  )(scores.reshape(NW * (1 + K), BPW))
  return loss[0, 0]


# speedup vs baseline: 2.2821x; 2.2821x over previous
import functools

import jax
import jax.numpy as jnp
from jax import lax
from jax.experimental import pallas as pl
from jax.experimental.pallas import tpu as pltpu
from jax.experimental.pallas import tpu_sc as plsc

NC = 2
NS = 16
L = 16
NW = NC * NS

B = 16384
K = 10
D = 64
VOCAB = 1000000

BPW = B // NW
CHUNK = 32
NCHUNK = BPW // CHUNK
NGRP = CHUNK // L
NSROWS = CHUNK * K
NSU = 4
NSUR = NSROWS // NSU

NBLK = VOCAB // 128
TPW = NBLK // NW
NEXTRA = NBLK - TPW * NW
TAILROW = VOCAB - 128


def _transpose_block(in_v, out_v, iota):
  for d0 in range(0, D, L):
    dvec = d0 + iota
    def cbody(cc, carry, dvec=dvec):
      c0 = cc * L
      def jbody(j, carry2):
        cperm = c0 + ((iota + j) & (L - 1))
        v = plsc.load_gather(in_v, [dvec, cperm])
        plsc.store_scatter(out_v, [cperm, dvec], v)
        return carry2
      return lax.fori_loop(0, L, jbody, carry, unroll=8)
    lax.fori_loop(0, 128 // L, cbody, 0)


def _tr_body(wt_hbm, tail_hbm, out_hbm,
             in0, in1, out0, out1, semi0, semi1, semo0, semo1):
  wid = lax.axis_index("s") * NC + lax.axis_index("c")
  iota = lax.iota(jnp.int32, L)
  ins = (in0, in1)
  outs = (out0, out1)
  isems = (semi0, semi1)
  osems = (semo0, semo1)

  def rd(j, b):
    pltpu.async_copy(wt_hbm.at[:, pl.ds(j * 128, 128)], ins[b], isems[b])

  def wr(j, b):
    pltpu.async_copy(outs[b], out_hbm.at[pl.ds(j * 128, 128)], osems[b])

  def wait_rd(b):
    pltpu.make_async_copy(wt_hbm.at[:, pl.ds(0, 128)], ins[b],
                          isems[b]).wait()

  def wait_wr(b):
    pltpu.make_async_copy(outs[b], out_hbm.at[pl.ds(0, 128)],
                          osems[b]).wait()

  jof = lambda t: wid + NW * t
  jclamp = lambda t: jnp.minimum(jof(t), NBLK - 1)

  rd(jof(0), 0)
  rd(jof(1), 1)
  for b in range(2):
    wait_rd(b)
    _transpose_block(ins[b], outs[b], iota)
    rd(jclamp(b + 2), b)
    wr(jof(b), b)

  def body(s, carry):
    for b in range(2):
      t = 2 * s + b
      wait_rd(b)
      wait_wr(b)
      _transpose_block(ins[b], outs[b], iota)
      rd(jclamp(t + 2), b)
      wr(jof(t), b)
    return carry

  lax.fori_loop(1, TPW // 2, body, 0)

  for b in range(2):
    wait_rd(b)
    wait_wr(b)

  @pl.when(wid < NEXTRA)
  def _():
    j = TPW * NW + wid
    pltpu.async_copy(wt_hbm.at[:, pl.ds(j * 128, 128)], ins[0],
                     isems[0]).wait()
    _transpose_block(ins[0], outs[0], iota)
    pltpu.async_copy(outs[0], out_hbm.at[pl.ds(j * 128, 128)],
                     osems[0]).wait()

  @pl.when(wid == NEXTRA)
  def _():
    pltpu.async_copy(tail_hbm, ins[1], isems[1]).wait()
    _transpose_block(ins[1], outs[1], iota)
    pltpu.async_copy(outs[1], out_hbm.at[pl.ds(TAILROW, 128)],
                     osems[1]).wait()


def _retile(wt, tail):
  mesh = plsc.VectorSubcoreMesh(core_axis_name="c", subcore_axis_name="s")
  return pl.kernel(
      _tr_body,
      out_type=jax.ShapeDtypeStruct((VOCAB, 2 * D), jnp.float32),
      mesh=mesh,
      compiler_params=pltpu.CompilerParams(needs_layout_passes=False),
      scratch_types=[
          pltpu.VMEM((D, 128), jnp.float32),
          pltpu.VMEM((D, 128), jnp.float32),
          pltpu.VMEM((128, 128), jnp.float32),
          pltpu.VMEM((128, 128), jnp.float32),
          pltpu.SemaphoreType.DMA,
          pltpu.SemaphoreType.DMA,
          pltpu.SemaphoreType.DMA,
          pltpu.SemaphoreType.DMA,
      ],
  )(wt, tail)


def _sc_body(cen_i, ctx_i, ns_i, wc_hbm, wx_hbm, out_hbm,
             icen_v, ictx_v, ins_v, score_v,
             c_rows0, c_rows1, x_rows0, x_rows1, n_rows0, n_rows1,
             sem0, sem1):
  wid = lax.axis_index("s") * NC + lax.axis_index("c")

  pltpu.sync_copy(cen_i.at[wid], icen_v)
  pltpu.sync_copy(ctx_i.at[wid], ictx_v)
  pltpu.sync_copy(ns_i.at[wid], ins_v)

  bufs = ((c_rows0, x_rows0, n_rows0, sem0),
          (c_rows1, x_rows1, n_rows1, sem1))

  def issue(g):
    c_b, x_b, n_b, sem = bufs[g % 2]
    cps = [
        pltpu.async_copy(wc_hbm.at[icen_v.at[pl.ds(g * CHUNK, CHUNK)]],
                         c_b, sem),
        pltpu.async_copy(wx_hbm.at[ictx_v.at[pl.ds(g * CHUNK, CHUNK)]],
                         x_b, sem),
    ]
    for u in range(NSU):
      cps.append(pltpu.async_copy(
          wx_hbm.at[ins_v.at[pl.ds(g * NSROWS + u * NSUR, NSUR)]],
          n_b.at[pl.ds(u * NSUR, NSUR)], sem))
    return cps

  iota = lax.iota(jnp.int32, L)
  pending = issue(0)

  for g in range(NCHUNK):
    nxt = issue(g + 1) if g + 1 < NCHUNK else None
    for cp in pending:
      cp.wait()
    pending = nxt

    c_b, x_b, n_b, _ = bufs[g % 2]
    for grp in range(NGRP):
      base = g * CHUNK + grp * L
      row = grp * L + iota
      nrow = [row * K + k for k in range(K)]

      def body(d, accs):
        dv = (d + iota) & (D - 1)
        cv = plsc.load_gather(c_b, [row, dv])
        xv = plsc.load_gather(x_b, [row, dv])
        new = [accs[0] + cv * xv]
        for k in range(K):
          nv = plsc.load_gather(n_b, [nrow[k], dv])
          new.append(accs[k + 1] + cv * nv)
        return tuple(new)

      accs = lax.fori_loop(
          0, D, body, tuple(jnp.zeros((L,), jnp.float32) for _ in range(K + 1)))

      score_v[0, pl.ds(base, L)] = -accs[0]
      for k in range(K):
        score_v[1 + k, pl.ds(base, L)] = accs[k + 1]

  pltpu.sync_copy(score_v, out_hbm.at[wid])


def _tc_body(s_ref, o_ref):
  x = s_ref[...]
  o_ref[0, 0] = jnp.sum(jnp.maximum(x, 0.0) +
                        jnp.log1p(jnp.exp(-jnp.abs(x))))


@jax.jit
def kernel(center, context, ns, W_center, W_context):
  cen_i = center.astype(jnp.int32).reshape(NW, BPW)
  ctx_i = context.astype(jnp.int32).reshape(NW, BPW)
  ns_i = ns.astype(jnp.int32).reshape(NW, BPW * K)

  wct = W_center.T
  wxt = W_context.T
  wc2 = _retile(wct, lax.slice(wct, (0, TAILROW), (D, VOCAB)))
  wx2 = _retile(wxt, lax.slice(wxt, (0, TAILROW), (D, VOCAB)))

  mesh = plsc.VectorSubcoreMesh(core_axis_name="c", subcore_axis_name="s")
  scores = pl.kernel(
      _sc_body,
      out_type=jax.ShapeDtypeStruct((NW, 1 + K, BPW), jnp.float32),
      mesh=mesh,
      compiler_params=pltpu.CompilerParams(needs_layout_passes=False),
      scratch_types=[
          pltpu.VMEM((BPW,), jnp.int32),
          pltpu.VMEM((BPW,), jnp.int32),
          pltpu.VMEM((BPW * K,), jnp.int32),
          pltpu.VMEM((1 + K, BPW), jnp.float32),
          pltpu.VMEM((CHUNK, 2 * D), jnp.float32),
          pltpu.VMEM((CHUNK, 2 * D), jnp.float32),
          pltpu.VMEM((CHUNK, 2 * D), jnp.float32),
          pltpu.VMEM((CHUNK, 2 * D), jnp.float32),
          pltpu.VMEM((NSROWS, 2 * D), jnp.float32),
          pltpu.VMEM((NSROWS, 2 * D), jnp.float32),
          pltpu.SemaphoreType.DMA,
          pltpu.SemaphoreType.DMA,
      ],
  )(cen_i, ctx_i, ns_i, wc2, wx2)

  loss = pl.pallas_call(
      _tc_body,
      out_shape=jax.ShapeDtypeStruct((1, 1), jnp.float32),
      out_specs=pl.BlockSpec(memory_space=pltpu.SMEM),
  )(scores.reshape(NW * (1 + K), BPW))
  return loss[0, 0]


# speedup vs baseline: 4.0894x; 1.7919x over previous
import functools

import jax
import jax.numpy as jnp
from jax import lax
from jax.experimental import pallas as pl
from jax.experimental.pallas import tpu as pltpu
from jax.experimental.pallas import tpu_sc as plsc

NC = 2
NS = 16
L = 16
NW = NC * NS

B = 16384
K = 10
D = 64
VOCAB = 1000000

BPW = B // NW
CHUNK = 32
NCHUNK = BPW // CHUNK
NGRP = CHUNK // L
NSROWS = CHUNK * K
NSU = 4
NSUR = NSROWS // NSU

NBLK = VOCAB // 128
TPW = NBLK // NW
NEXTRA = NBLK - TPW * NW
TAILROW = VOCAB - 128


def _transpose_block(in_v, out_v, iota):
  for d0 in range(0, D, L):
    dvec = d0 + iota

    @plsc.parallel_loop(0, 128, unroll=8)
    def _jc(t, dvec=dvec):
      cperm = (t & ~(L - 1)) + ((iota + t) & (L - 1))
      v = plsc.load_gather(in_v, [dvec, cperm])
      plsc.store_scatter(out_v, [cperm, dvec], v)


def _tr_body(wt_hbm, tail_hbm, out_hbm,
             in0, in1, out0, out1, semi0, semi1, semo0, semo1):
  wid = lax.axis_index("s") * NC + lax.axis_index("c")
  iota = lax.iota(jnp.int32, L)
  ins = (in0, in1)
  outs = (out0, out1)
  isems = (semi0, semi1)
  osems = (semo0, semo1)

  def rd(j, b):
    pltpu.async_copy(wt_hbm.at[:, pl.ds(j * 128, 128)], ins[b], isems[b])

  def wr(j, b):
    pltpu.async_copy(outs[b], out_hbm.at[pl.ds(j * 128, 128)], osems[b])

  def wait_rd(b):
    pltpu.make_async_copy(wt_hbm.at[:, pl.ds(0, 128)], ins[b],
                          isems[b]).wait()

  def wait_wr(b):
    pltpu.make_async_copy(outs[b], out_hbm.at[pl.ds(0, 128)],
                          osems[b]).wait()

  jof = lambda t: wid + NW * t
  jclamp = lambda t: jnp.minimum(jof(t), NBLK - 1)

  rd(jof(0), 0)
  rd(jof(1), 1)
  for b in range(2):
    wait_rd(b)
    _transpose_block(ins[b], outs[b], iota)
    rd(jclamp(b + 2), b)
    wr(jof(b), b)

  def body(s, carry):
    for b in range(2):
      t = 2 * s + b
      wait_rd(b)
      wait_wr(b)
      _transpose_block(ins[b], outs[b], iota)
      rd(jclamp(t + 2), b)
      wr(jof(t), b)
    return carry

  lax.fori_loop(1, TPW // 2, body, 0)

  for b in range(2):
    wait_rd(b)
    wait_wr(b)

  @pl.when(wid < NEXTRA)
  def _():
    j = TPW * NW + wid
    pltpu.async_copy(wt_hbm.at[:, pl.ds(j * 128, 128)], ins[0],
                     isems[0]).wait()
    _transpose_block(ins[0], outs[0], iota)
    pltpu.async_copy(outs[0], out_hbm.at[pl.ds(j * 128, 128)],
                     osems[0]).wait()

  @pl.when(wid == NEXTRA)
  def _():
    pltpu.async_copy(tail_hbm, ins[1], isems[1]).wait()
    _transpose_block(ins[1], outs[1], iota)
    pltpu.async_copy(outs[1], out_hbm.at[pl.ds(TAILROW, 128)],
                     osems[1]).wait()


def _retile(wt, tail):
  mesh = plsc.VectorSubcoreMesh(core_axis_name="c", subcore_axis_name="s")
  return pl.kernel(
      _tr_body,
      out_type=jax.ShapeDtypeStruct((VOCAB, 2 * D), jnp.float32),
      mesh=mesh,
      compiler_params=pltpu.CompilerParams(needs_layout_passes=False),
      scratch_types=[
          pltpu.VMEM((D, 128), jnp.float32),
          pltpu.VMEM((D, 128), jnp.float32),
          pltpu.VMEM((128, 128), jnp.float32),
          pltpu.VMEM((128, 128), jnp.float32),
          pltpu.SemaphoreType.DMA,
          pltpu.SemaphoreType.DMA,
          pltpu.SemaphoreType.DMA,
          pltpu.SemaphoreType.DMA,
      ],
  )(wt, tail)


def _sc_body(cen_i, ctx_i, ns_i, wc_hbm, wx_hbm, out_hbm,
             icen_v, ictx_v, ins_v, score_v,
             c_rows0, c_rows1, x_rows0, x_rows1, n_rows0, n_rows1,
             sem0, sem1):
  wid = lax.axis_index("s") * NC + lax.axis_index("c")

  pltpu.sync_copy(cen_i.at[wid], icen_v)
  pltpu.sync_copy(ctx_i.at[wid], ictx_v)
  pltpu.sync_copy(ns_i.at[wid], ins_v)

  bufs = ((c_rows0, x_rows0, n_rows0, sem0),
          (c_rows1, x_rows1, n_rows1, sem1))

  def issue(g):
    c_b, x_b, n_b, sem = bufs[g % 2]
    cps = [
        pltpu.async_copy(wc_hbm.at[icen_v.at[pl.ds(g * CHUNK, CHUNK)]],
                         c_b, sem),
        pltpu.async_copy(wx_hbm.at[ictx_v.at[pl.ds(g * CHUNK, CHUNK)]],
                         x_b, sem),
    ]
    for u in range(NSU):
      cps.append(pltpu.async_copy(
          wx_hbm.at[ins_v.at[pl.ds(g * NSROWS + u * NSUR, NSUR)]],
          n_b.at[pl.ds(u * NSUR, NSUR)], sem))
    return cps

  iota = lax.iota(jnp.int32, L)
  pending = issue(0)

  for g in range(NCHUNK):
    nxt = issue(g + 1) if g + 1 < NCHUNK else None
    for cp in pending:
      cp.wait()
    pending = nxt

    c_b, x_b, n_b, _ = bufs[g % 2]
    for grp in range(NGRP):
      base = g * CHUNK + grp * L
      row = grp * L + iota
      nrow = [row * K + k for k in range(K)]

      def body(d, accs):
        dv = (d + iota) & (D - 1)
        cv = plsc.load_gather(c_b, [row, dv])
        xv = plsc.load_gather(x_b, [row, dv])
        new = [accs[0] + cv * xv]
        for k in range(K):
          nv = plsc.load_gather(n_b, [nrow[k], dv])
          new.append(accs[k + 1] + cv * nv)
        return tuple(new)

      accs = lax.fori_loop(
          0, D, body, tuple(jnp.zeros((L,), jnp.float32) for _ in range(K + 1)))

      score_v[0, pl.ds(base, L)] = -accs[0]
      for k in range(K):
        score_v[1 + k, pl.ds(base, L)] = accs[k + 1]

  pltpu.sync_copy(score_v, out_hbm.at[wid])


def _tc_body(s_ref, o_ref):
  x = s_ref[...]
  o_ref[0, 0] = jnp.sum(jnp.maximum(x, 0.0) +
                        jnp.log1p(jnp.exp(-jnp.abs(x))))


@jax.jit
def kernel(center, context, ns, W_center, W_context):
  cen_i = center.astype(jnp.int32).reshape(NW, BPW)
  ctx_i = context.astype(jnp.int32).reshape(NW, BPW)
  ns_i = ns.astype(jnp.int32).reshape(NW, BPW * K)

  wct = W_center.T
  wxt = W_context.T
  wc2 = _retile(wct, lax.slice(wct, (0, TAILROW), (D, VOCAB)))
  wx2 = _retile(wxt, lax.slice(wxt, (0, TAILROW), (D, VOCAB)))

  mesh = plsc.VectorSubcoreMesh(core_axis_name="c", subcore_axis_name="s")
  scores = pl.kernel(
      _sc_body,
      out_type=jax.ShapeDtypeStruct((NW, 1 + K, BPW), jnp.float32),
      mesh=mesh,
      compiler_params=pltpu.CompilerParams(needs_layout_passes=False),
      scratch_types=[
          pltpu.VMEM((BPW,), jnp.int32),
          pltpu.VMEM((BPW,), jnp.int32),
          pltpu.VMEM((BPW * K,), jnp.int32),
          pltpu.VMEM((1 + K, BPW), jnp.float32),
          pltpu.VMEM((CHUNK, 2 * D), jnp.float32),
          pltpu.VMEM((CHUNK, 2 * D), jnp.float32),
          pltpu.VMEM((CHUNK, 2 * D), jnp.float32),
          pltpu.VMEM((CHUNK, 2 * D), jnp.float32),
          pltpu.VMEM((NSROWS, 2 * D), jnp.float32),
          pltpu.VMEM((NSROWS, 2 * D), jnp.float32),
          pltpu.SemaphoreType.DMA,
          pltpu.SemaphoreType.DMA,
      ],
  )(cen_i, ctx_i, ns_i, wc2, wx2)

  loss = pl.pallas_call(
      _tc_body,
      out_shape=jax.ShapeDtypeStruct((1, 1), jnp.float32),
      out_specs=pl.BlockSpec(memory_space=pltpu.SMEM),
  )(scores.reshape(NW * (1 + K), BPW))
  return loss[0, 0]


# speedup vs baseline: 4.1258x; 1.0089x over previous
import functools

import jax
import jax.numpy as jnp
from jax import lax
from jax.experimental import pallas as pl
from jax.experimental.pallas import tpu as pltpu
from jax.experimental.pallas import tpu_sc as plsc

NC = 2
NS = 16
L = 16
NW = NC * NS

B = 16384
K = 10
D = 64
VOCAB = 1000000

BPW = B // NW
CHUNK = 32
NCHUNK = BPW // CHUNK
NGRP = CHUNK // L
NSROWS = CHUNK * K
NSU = 4
NSUR = NSROWS // NSU

NBLK = VOCAB // 128
TPW = NBLK // NW
NEXTRA = NBLK - TPW * NW
TAILROW = VOCAB - 128


def _transpose_block(in_v, out_v, iota):
  @plsc.parallel_loop(0, (D // L) * 128, unroll=16)
  def _jc(t):
    tc = t & 127
    dvec = ((t >> 7) << 4) + iota
    cperm = (tc & ~(L - 1)) + ((iota + tc) & (L - 1))
    v = plsc.load_gather(in_v, [dvec, cperm])
    plsc.store_scatter(out_v, [cperm, dvec], v)


def _tr_body(wt_hbm, tail_hbm, out_hbm,
             in0, in1, out0, out1, semi0, semi1, semo0, semo1):
  wid = lax.axis_index("s") * NC + lax.axis_index("c")
  iota = lax.iota(jnp.int32, L)
  ins = (in0, in1)
  outs = (out0, out1)
  isems = (semi0, semi1)
  osems = (semo0, semo1)

  def rd(j, b):
    pltpu.async_copy(wt_hbm.at[:, pl.ds(j * 128, 128)], ins[b], isems[b])

  def wr(j, b):
    pltpu.async_copy(outs[b], out_hbm.at[pl.ds(j * 128, 128)], osems[b])

  def wait_rd(b):
    pltpu.make_async_copy(wt_hbm.at[:, pl.ds(0, 128)], ins[b],
                          isems[b]).wait()

  def wait_wr(b):
    pltpu.make_async_copy(outs[b], out_hbm.at[pl.ds(0, 128)],
                          osems[b]).wait()

  jof = lambda t: wid + NW * t
  jclamp = lambda t: jnp.minimum(jof(t), NBLK - 1)

  rd(jof(0), 0)
  rd(jof(1), 1)
  for b in range(2):
    wait_rd(b)
    _transpose_block(ins[b], outs[b], iota)
    rd(jclamp(b + 2), b)
    wr(jof(b), b)

  def body(s, carry):
    for b in range(2):
      t = 2 * s + b
      wait_rd(b)
      wait_wr(b)
      _transpose_block(ins[b], outs[b], iota)
      rd(jclamp(t + 2), b)
      wr(jof(t), b)
    return carry

  lax.fori_loop(1, TPW // 2, body, 0)

  for b in range(2):
    wait_rd(b)
    wait_wr(b)

  @pl.when(wid < NEXTRA)
  def _():
    j = TPW * NW + wid
    pltpu.async_copy(wt_hbm.at[:, pl.ds(j * 128, 128)], ins[0],
                     isems[0]).wait()
    _transpose_block(ins[0], outs[0], iota)
    pltpu.async_copy(outs[0], out_hbm.at[pl.ds(j * 128, 128)],
                     osems[0]).wait()

  @pl.when(wid == NEXTRA)
  def _():
    pltpu.async_copy(tail_hbm, ins[1], isems[1]).wait()
    _transpose_block(ins[1], outs[1], iota)
    pltpu.async_copy(outs[1], out_hbm.at[pl.ds(TAILROW, 128)],
                     osems[1]).wait()


def _retile(wt, tail):
  mesh = plsc.VectorSubcoreMesh(core_axis_name="c", subcore_axis_name="s")
  return pl.kernel(
      _tr_body,
      out_type=jax.ShapeDtypeStruct((VOCAB, 2 * D), jnp.float32),
      mesh=mesh,
      compiler_params=pltpu.CompilerParams(needs_layout_passes=False),
      scratch_types=[
          pltpu.VMEM((D, 128), jnp.float32),
          pltpu.VMEM((D, 128), jnp.float32),
          pltpu.VMEM((128, 128), jnp.float32),
          pltpu.VMEM((128, 128), jnp.float32),
          pltpu.SemaphoreType.DMA,
          pltpu.SemaphoreType.DMA,
          pltpu.SemaphoreType.DMA,
          pltpu.SemaphoreType.DMA,
      ],
  )(wt, tail)


def _sc_body(cen_i, ctx_i, ns_i, wc_hbm, wx_hbm, out_hbm,
             icen_v, ictx_v, ins_v, score_v,
             c_rows0, c_rows1, x_rows0, x_rows1, n_rows0, n_rows1,
             sem0, sem1):
  wid = lax.axis_index("s") * NC + lax.axis_index("c")

  pltpu.sync_copy(cen_i.at[wid], icen_v)
  pltpu.sync_copy(ctx_i.at[wid], ictx_v)
  pltpu.sync_copy(ns_i.at[wid], ins_v)

  bufs = ((c_rows0, x_rows0, n_rows0, sem0),
          (c_rows1, x_rows1, n_rows1, sem1))

  def issue(g):
    c_b, x_b, n_b, sem = bufs[g % 2]
    cps = [
        pltpu.async_copy(wc_hbm.at[icen_v.at[pl.ds(g * CHUNK, CHUNK)]],
                         c_b, sem),
        pltpu.async_copy(wx_hbm.at[ictx_v.at[pl.ds(g * CHUNK, CHUNK)]],
                         x_b, sem),
    ]
    for u in range(NSU):
      cps.append(pltpu.async_copy(
          wx_hbm.at[ins_v.at[pl.ds(g * NSROWS + u * NSUR, NSUR)]],
          n_b.at[pl.ds(u * NSUR, NSUR)], sem))
    return cps

  iota = lax.iota(jnp.int32, L)
  pending = issue(0)

  for g in range(NCHUNK):
    nxt = issue(g + 1) if g + 1 < NCHUNK else None
    for cp in pending:
      cp.wait()
    pending = nxt

    c_b, x_b, n_b, _ = bufs[g % 2]
    for grp in range(NGRP):
      base = g * CHUNK + grp * L
      row = grp * L + iota
      nrow = [row * K + k for k in range(K)]

      def body(d, accs):
        dv = (d + iota) & (D - 1)
        cv = plsc.load_gather(c_b, [row, dv])
        xv = plsc.load_gather(x_b, [row, dv])
        new = [accs[0] + cv * xv]
        for k in range(K):
          nv = plsc.load_gather(n_b, [nrow[k], dv])
          new.append(accs[k + 1] + cv * nv)
        return tuple(new)

      accs = lax.fori_loop(
          0, D, body, tuple(jnp.zeros((L,), jnp.float32) for _ in range(K + 1)))

      score_v[0, pl.ds(base, L)] = -accs[0]
      for k in range(K):
        score_v[1 + k, pl.ds(base, L)] = accs[k + 1]

  pltpu.sync_copy(score_v, out_hbm.at[wid])


def _tc_body(s_ref, o_ref):
  x = s_ref[...]
  o_ref[0, 0] = jnp.sum(jnp.maximum(x, 0.0) +
                        jnp.log1p(jnp.exp(-jnp.abs(x))))


@jax.jit
def kernel(center, context, ns, W_center, W_context):
  cen_i = center.astype(jnp.int32).reshape(NW, BPW)
  ctx_i = context.astype(jnp.int32).reshape(NW, BPW)
  ns_i = ns.astype(jnp.int32).reshape(NW, BPW * K)

  wct = W_center.T
  wxt = W_context.T
  wc2 = _retile(wct, lax.slice(wct, (0, TAILROW), (D, VOCAB)))
  wx2 = _retile(wxt, lax.slice(wxt, (0, TAILROW), (D, VOCAB)))

  mesh = plsc.VectorSubcoreMesh(core_axis_name="c", subcore_axis_name="s")
  scores = pl.kernel(
      _sc_body,
      out_type=jax.ShapeDtypeStruct((NW, 1 + K, BPW), jnp.float32),
      mesh=mesh,
      compiler_params=pltpu.CompilerParams(needs_layout_passes=False),
      scratch_types=[
          pltpu.VMEM((BPW,), jnp.int32),
          pltpu.VMEM((BPW,), jnp.int32),
          pltpu.VMEM((BPW * K,), jnp.int32),
          pltpu.VMEM((1 + K, BPW), jnp.float32),
          pltpu.VMEM((CHUNK, 2 * D), jnp.float32),
          pltpu.VMEM((CHUNK, 2 * D), jnp.float32),
          pltpu.VMEM((CHUNK, 2 * D), jnp.float32),
          pltpu.VMEM((CHUNK, 2 * D), jnp.float32),
          pltpu.VMEM((NSROWS, 2 * D), jnp.float32),
          pltpu.VMEM((NSROWS, 2 * D), jnp.float32),
          pltpu.SemaphoreType.DMA,
          pltpu.SemaphoreType.DMA,
      ],
  )(cen_i, ctx_i, ns_i, wc2, wx2)

  loss = pl.pallas_call(
      _tc_body,
      out_shape=jax.ShapeDtypeStruct((1, 1), jnp.float32),
      out_specs=pl.BlockSpec(memory_space=pltpu.SMEM),
  )(scores.reshape(NW * (1 + K), BPW))
  return loss[0, 0]
